# baseline (speedup 1.0000x reference)
import functools

import jax
import jax.numpy as jnp
from jax import lax
from jax.experimental import pallas as pl
from jax.experimental.pallas import tpu as pltpu
from jax.experimental.pallas import tpu_sc as plsc

NC = 2
NS = 16
NL = 16
NW = NC * NS


def _mesh():
    return plsc.VectorSubcoreMesh(
        core_axis_name="c", subcore_axis_name="s", num_cores=NC, num_subcores=NS
    )


_SC_PARAMS = pltpu.CompilerParams(
    needs_layout_passes=False, use_tc_tiling_on_sc=False
)


def _wid():
    return lax.axis_index("s") * NC + lax.axis_index("c")


def _fire(src, dst, sem):
    pltpu.make_async_copy(src, dst, sem).start()


def _drain(src, dst, sem):
    pltpu.make_async_copy(src, dst, sem).wait()


def _compose_kernel(U2, U1, S):
    RPT = U1 // NW
    M = RPT * S

    @functools.partial(
        pl.kernel,
        out_type=(
            jax.ShapeDtypeStruct((U1 * S,), jnp.int32),
            jax.ShapeDtypeStruct((U1,), jnp.int32),
        ),
        mesh=_mesh(),
        compiler_params=_SC_PARAMS,
        scratch_types=[
            pltpu.VMEM((U2,), jnp.int32),
            pltpu.VMEM((M,), jnp.int32),
            pltpu.VMEM((RPT,), jnp.int32),
        ],
    )
    def k(nodes_hbm, neigh2_hbm, cur1_hbm, comp2_hbm, compc_hbm, nodes_v, buf, cbuf):
        base = _wid() * RPT
        pltpu.sync_copy(nodes_hbm, nodes_v)
        pltpu.sync_copy(neigh2_hbm.at[pl.ds(base * S, M)], buf)
        pltpu.sync_copy(cur1_hbm.at[pl.ds(base, RPT)], cbuf)

        UNR = 4

        def body(i, _):
            for j in range(UNR):
                off = pl.multiple_of(i * NL * UNR + j * NL, NL)
                v = buf[pl.ds(off, NL)]
                buf[pl.ds(off, NL)] = plsc.load_gather(nodes_v, [v])
            return 0

        lax.fori_loop(0, M // (NL * UNR), body, 0)

        def cbody(i, _):
            for j in range(UNR):
                off = pl.multiple_of(i * NL * UNR + j * NL, NL)
                v = cbuf[pl.ds(off, NL)]
                cbuf[pl.ds(off, NL)] = plsc.load_gather(nodes_v, [v])
            return 0

        lax.fori_loop(0, RPT // (NL * UNR), cbody, 0)

        pltpu.sync_copy(buf, comp2_hbm.at[pl.ds(base * S, M)])
        pltpu.sync_copy(cbuf, compc_hbm.at[pl.ds(base, RPT)])

    return k


def _sum16_row(gbuf, slot, r, S, W_, obuf, obs, orow):
    fmt = plsc.PackFormat.INTERLEAVED
    for c in range(W_ // 16):
        cs = pl.ds(c * 16, 16)
        v = plsc.bitcast(gbuf[slot, r * S, cs], jnp.bfloat16)
        a0, a1 = plsc.unpack(v, format=fmt)
        for s in range(1, S):
            v = plsc.bitcast(gbuf[slot, r * S + s, cs], jnp.bfloat16)
            b0, b1 = plsc.unpack(v, format=fmt)
            a0 = a0 + b0
            a1 = a1 + b1
        obuf[obs, orow, cs] = plsc.bitcast(plsc.pack(a0, a1, format=fmt), jnp.int32)


def _gather_sum_kernel(W_, U1, S):
    RPT = U1 // NW
    GR = 128
    NG = RPT * S // GR
    RG = GR // S
    FL = 16
    OB = FL * RG
    NX = RPT // GR

    @functools.partial(
        pl.kernel,
        out_type=(
            jax.ShapeDtypeStruct((U1, W_), jnp.int32),
            jax.ShapeDtypeStruct((U1, W_), jnp.int32),
        ),
        mesh=_mesh(),
        compiler_params=_SC_PARAMS,
        scratch_types=[
            pltpu.VMEM((RPT * S,), jnp.int32),
            pltpu.VMEM((RPT,), jnp.int32),
            pltpu.VMEM((2, GR, W_), jnp.int32),
            pltpu.VMEM((1, OB, W_), jnp.int32),
            pltpu.SemaphoreType.DMA,
            pltpu.SemaphoreType.DMA,
        ],
    )
    def k(feat_hbm, comp2_hbm, compc_hbm, sum_hbm, x_hbm, cidx, ccur, gbuf, obuf,
          semA, semB):
        base = _wid() * RPT
        pltpu.sync_copy(comp2_hbm.at[pl.ds(base * S, RPT * S)], cidx)
        pltpu.sync_copy(compc_hbm.at[pl.ds(base, RPT)], ccur)
        sems = (semA, semB)

        def xsrc(i):
            off = pl.multiple_of(i * GR, GR)
            return feat_hbm.at[ccur.at[pl.ds(off, GR)]]

        _fire(xsrc(0), gbuf.at[0], sems[0])
        for i in range(NX):
            slot = i % 2
            if i + 1 < NX:
                _fire(xsrc(i + 1), gbuf.at[1 - slot], sems[1 - slot])
            _drain(xsrc(i), gbuf.at[slot], sems[slot])
            pltpu.sync_copy(gbuf.at[slot], x_hbm.at[pl.ds(base + i * GR, GR)])

        def gsrc(g):
            off = pl.multiple_of(g * GR, GR)
            return feat_hbm.at[cidx.at[pl.ds(off, GR)]]

        def gstart(g, slot):
            _fire(gsrc(g), gbuf.at[slot], sems[slot])

        gstart(0, 0)

        def pair_body(gp, _):
            for kk in range(2):
                g = gp * 2 + kk
                slot = kk

                @pl.when(g + 1 < NG)
                def _():
                    gstart(g + 1, 1 - slot)

                _drain(gsrc(g), gbuf.at[slot], sems[slot])

                def red_body(r, _):
                    _sum16_row(gbuf, slot, r, S, W_, obuf, 0, (g % FL) * RG + r)
                    return 0

                lax.fori_loop(0, RG, red_body, 0)

                @pl.when(g % FL == FL - 1)
                def _():
                    pltpu.sync_copy(
                        obuf.at[0], sum_hbm.at[pl.ds(base + (g // FL) * OB, OB)]
                    )
            return 0

        lax.fori_loop(0, NG // 2, pair_body, 0)

    return k


def _gather_sum_small_kernel(W_, B_, S):
    RB = B_ // NW
    M = RB * S
    GR = 128
    NG = M // GR
    RG = GR // S

    @functools.partial(
        pl.kernel,
        out_type=(
            jax.ShapeDtypeStruct((B_, W_), jnp.int32),
            jax.ShapeDtypeStruct((B_, W_), jnp.int32),
        ),
        mesh=_mesh(),
        compiler_params=_SC_PARAMS,
        scratch_types=[
            pltpu.VMEM((M,), jnp.int32),
            pltpu.VMEM((RB,), jnp.int32),
            pltpu.VMEM((2, GR, W_), jnp.int32),
            pltpu.VMEM((1, RB, W_), jnp.int32),
            pltpu.VMEM((RB, W_), jnp.int32),
            pltpu.SemaphoreType.DMA,
            pltpu.SemaphoreType.DMA,
        ],
    )
    def k(h_hbm, n1_hbm, cur2_hbm, sum_hbm, x_hbm, cidx, ccur, gbuf, obuf, xg,
          semA, semB):
        base = _wid() * RB
        pltpu.sync_copy(n1_hbm.at[pl.ds(base * S, M)], cidx)
        pltpu.sync_copy(cur2_hbm.at[pl.ds(base, RB)], ccur)
        sems = (semA, semB)

        _fire(h_hbm.at[ccur], xg, semA)

        def gsrc(g):
            off = pl.multiple_of(g * GR, GR)
            return h_hbm.at[cidx.at[pl.ds(off, GR)]]

        _fire(gsrc(0), gbuf.at[0], sems[1])
        _drain(h_hbm.at[ccur], xg, semA)
        pltpu.sync_copy(xg, x_hbm.at[pl.ds(base, RB)])

        for g in range(NG):
            slot = g % 2
            if g + 1 < NG:
                _fire(gsrc(g + 1), gbuf.at[1 - slot], sems[g % 2])
            _drain(gsrc(g), gbuf.at[slot], sems[1 - slot])

            def red_body(r, _):
                _sum16_row(gbuf, slot, r, S, W_, obuf, 0, g * RG + r)
                return 0

            lax.fori_loop(0, RG, red_body, 0)

        pltpu.sync_copy(obuf.at[0], sum_hbm.at[pl.ds(base, RB)])

    return k


def _fused_linear(x, s, Wa, Wb, blk, out_dtype):
    R, Dm = x.shape
    Hm = Wa.shape[1]

    def body(x_ref, s_ref, wa_ref, wb_ref, o_ref):
        acc = jnp.dot(x_ref[...], wa_ref[...], preferred_element_type=jnp.float32)
        acc = acc + jnp.dot(s_ref[...], wb_ref[...], preferred_element_type=jnp.float32)
        o_ref[...] = jnp.maximum(acc, 0.0).astype(out_dtype)

    return pl.pallas_call(
        body,
        grid=(R // blk,),
        in_specs=[
            pl.BlockSpec((blk, Dm), lambda i: (i, 0)),
            pl.BlockSpec((blk, Dm), lambda i: (i, 0)),
            pl.BlockSpec((Dm, Hm), lambda i: (0, 0)),
            pl.BlockSpec((Dm, Hm), lambda i: (0, 0)),
        ],
        out_specs=pl.BlockSpec((blk, Hm), lambda i: (i, 0)),
        out_shape=jax.ShapeDtypeStruct((R, Hm), out_dtype),
    )(x, s, Wa, Wb)


def kernel(in_features, nodes_l0, neigh2, cur1, neigh1, cur2, W0, W1):
    N_, D_ = in_features.shape
    U2 = nodes_l0.shape[0]
    U1, S = neigh2.shape
    B_ = cur2.shape[0]
    H_ = W0.shape[1]

    def to_words(a):
        return lax.bitcast_convert_type(a.reshape(a.shape[0], -1, 2), jnp.int32)

    def from_words(w):
        return lax.bitcast_convert_type(w, jnp.bfloat16).reshape(w.shape[0], -1)

    featw = to_words(in_features.astype(jnp.bfloat16))
    nodes_l0 = nodes_l0.astype(jnp.int32)
    neigh2f = neigh2.astype(jnp.int32).reshape(U1 * S)
    cur1 = cur1.astype(jnp.int32)
    neigh1f = neigh1.astype(jnp.int32).reshape(B_ * S)
    cur2 = cur2.astype(jnp.int32)
    W0a = W0[:D_].astype(jnp.bfloat16)
    W0b = (W0[D_:] * (1.0 / S)).astype(jnp.bfloat16)
    W1a = W1[:H_].astype(jnp.bfloat16)
    W1b = (W1[H_:] * (1.0 / S)).astype(jnp.bfloat16)

    comp2, compc = _compose_kernel(U2, U1, S)(nodes_l0, neigh2f, cur1)
    sum0w, x0w = _gather_sum_kernel(D_ // 2, U1, S)(featw, comp2, compc)
    h1 = _fused_linear(from_words(x0w), from_words(sum0w), W0a, W0b, 512,
                       jnp.bfloat16)
    sum1w, x1w = _gather_sum_small_kernel(H_ // 2, B_, S)(to_words(h1), neigh1f,
                                                          cur2)
    out = _fused_linear(from_words(x1w), from_words(sum1w), W1a, W1b, 512,
                        jnp.float32)
    return out


# speedup vs baseline: 1.5848x; 1.5848x over previous
import functools

import jax
import jax.numpy as jnp
import numpy as np
from jax import lax
from jax.experimental import pallas as pl
from jax.experimental.pallas import tpu as pltpu
from jax.experimental.pallas import tpu_sc as plsc

NC = 2
NS = 16
NL = 16
NW = NC * NS


def _mesh():
    return plsc.VectorSubcoreMesh(
        core_axis_name="c", subcore_axis_name="s", num_cores=NC, num_subcores=NS
    )


_SC_PARAMS = pltpu.CompilerParams(needs_layout_passes=False)
_SC_PARAMS_NT = pltpu.CompilerParams(
    needs_layout_passes=False, use_tc_tiling_on_sc=False
)


def _wid():
    return lax.axis_index("s") * NC + lax.axis_index("c")


def _fire(src, dst, sem):
    pltpu.make_async_copy(src, dst, sem).start()


def _drain(src, dst, sem):
    pltpu.make_async_copy(src, dst, sem).wait()


def _compose_kernel(U2, U1, S):
    RPT = U1 // NW
    M = RPT * S

    @functools.partial(
        pl.kernel,
        out_type=(
            jax.ShapeDtypeStruct((U1 * S,), jnp.int32),
            jax.ShapeDtypeStruct((U1,), jnp.int32),
        ),
        mesh=_mesh(),
        compiler_params=_SC_PARAMS,
        scratch_types=[
            pltpu.VMEM((U2,), jnp.int32),
            pltpu.VMEM((M,), jnp.int32),
            pltpu.VMEM((RPT,), jnp.int32),
        ],
    )
    def k(nodes_hbm, neigh2_hbm, cur1_hbm, comp2_hbm, compc_hbm, nodes_v, buf, cbuf):
        base = _wid() * RPT
        pltpu.sync_copy(nodes_hbm, nodes_v)
        pltpu.sync_copy(neigh2_hbm.at[pl.ds(base * S, M)], buf)
        pltpu.sync_copy(cur1_hbm.at[pl.ds(base, RPT)], cbuf)

        UNR = 4

        def body(i, _):
            for j in range(UNR):
                off = pl.multiple_of(i * NL * UNR + j * NL, NL)
                v = buf[pl.ds(off, NL)]
                buf[pl.ds(off, NL)] = plsc.load_gather(nodes_v, [v])
            return 0

        lax.fori_loop(0, M // (NL * UNR), body, 0)

        def cbody(i, _):
            for j in range(UNR):
                off = pl.multiple_of(i * NL * UNR + j * NL, NL)
                v = cbuf[pl.ds(off, NL)]
                cbuf[pl.ds(off, NL)] = plsc.load_gather(nodes_v, [v])
            return 0

        lax.fori_loop(0, RPT // (NL * UNR), cbody, 0)

        pltpu.sync_copy(buf, comp2_hbm.at[pl.ds(base * S, M)])
        pltpu.sync_copy(cbuf, compc_hbm.at[pl.ds(base, RPT)])

    return k


def _gather_sum_kernel(W_, U1, S):
    D2 = 2 * W_
    RPT = U1 // NW
    GR = 128
    NG = RPT * S // GR
    RG = GR // S
    FL = 16
    OB = FL * RG
    NX = RPT // GR
    fmt = plsc.PackFormat.INTERLEAVED

    @functools.partial(
        pl.kernel,
        out_type=(
            jax.ShapeDtypeStruct((U1, D2), jnp.float32),
            jax.ShapeDtypeStruct((U1, D2), jnp.float32),
        ),
        mesh=_mesh(),
        compiler_params=_SC_PARAMS_NT,
        scratch_types=[
            pltpu.VMEM((RPT * S,), jnp.int32),
            pltpu.VMEM((RPT,), jnp.int32),
            pltpu.VMEM((2, GR, W_), jnp.int32),
            pltpu.VMEM((1, OB, D2), jnp.float32),
            pltpu.VMEM((2, GR, D2), jnp.float32),
            pltpu.SemaphoreType.DMA,
            pltpu.SemaphoreType.DMA,
            pltpu.SemaphoreType.DMA,
        ],
    )
    def k(feat_hbm, comp2_hbm, compc_hbm, sum_hbm, x_hbm, cidx, ccur, gbuf, obuf,
          xobuf, semA, semB, semX):
        base = _wid() * RPT
        pltpu.sync_copy(comp2_hbm.at[pl.ds(base * S, RPT * S)], cidx)
        pltpu.sync_copy(compc_hbm.at[pl.ds(base, RPT)], ccur)
        sems = (semA, semB)

        def xsrc(i):
            off = pl.multiple_of(i * GR, GR)
            return feat_hbm.at[ccur.at[pl.ds(off, GR)]]

        def xdst(i):
            return x_hbm.at[pl.ds(base + i * GR, GR)]

        _fire(xsrc(0), gbuf.at[0], sems[0])
        for i in range(NX):
            slot = i % 2
            if i + 1 < NX:
                _fire(xsrc(i + 1), gbuf.at[1 - slot], sems[1 - slot])
            _drain(xsrc(i), gbuf.at[slot], sems[slot])
            if i >= 2:
                _drain(xobuf.at[slot], xdst(i - 2), semX)

            def xw_body(r, _):
                for c in range(W_ // 16):
                    v = plsc.bitcast(gbuf[slot, r, pl.ds(c * 16, 16)],
                                     jnp.bfloat16)
                    a0, a1 = plsc.unpack(v, format=fmt)
                    xobuf[slot, r, pl.ds(c * 32, 16)] = a0
                    xobuf[slot, r, pl.ds(c * 32 + 16, 16)] = a1
                return 0

            lax.fori_loop(0, GR, xw_body, 0)
            _fire(xobuf.at[slot], xdst(i), semX)
        _drain(xobuf.at[(NX - 2) % 2], xdst(NX - 2), semX)
        _drain(xobuf.at[(NX - 1) % 2], xdst(NX - 1), semX)

        def gsrc(g):
            off = pl.multiple_of(g * GR, GR)
            return feat_hbm.at[cidx.at[pl.ds(off, GR)]]

        def gstart(g, slot):
            _fire(gsrc(g), gbuf.at[slot], sems[slot])

        gstart(0, 0)

        def pair_body(gp, _):
            for kk in range(2):
                g = gp * 2 + kk
                slot = kk

                @pl.when(g + 1 < NG)
                def _():
                    gstart(g + 1, 1 - slot)

                _drain(gsrc(g), gbuf.at[slot], sems[slot])

                def red_body(r, _):
                    orow = (g % FL) * RG + r
                    for c in range(W_ // 16):
                        cs = pl.ds(c * 16, 16)
                        v = plsc.bitcast(gbuf[slot, r * S, cs], jnp.bfloat16)
                        a0, a1 = plsc.unpack(v, format=fmt)
                        for s in range(1, S):
                            v = plsc.bitcast(gbuf[slot, r * S + s, cs],
                                             jnp.bfloat16)
                            b0, b1 = plsc.unpack(v, format=fmt)
                            a0 = a0 + b0
                            a1 = a1 + b1
                        obuf[0, orow, pl.ds(c * 32, 16)] = a0
                        obuf[0, orow, pl.ds(c * 32 + 16, 16)] = a1
                    return 0

                lax.fori_loop(0, RG, red_body, 0)

                @pl.when(g % FL == FL - 1)
                def _():
                    pltpu.sync_copy(
                        obuf.at[0], sum_hbm.at[pl.ds(base + (g // FL) * OB, OB)]
                    )
            return 0

        lax.fori_loop(0, NG // 2, pair_body, 0)

    return k


def _gather_sum_small_kernel(H_, B_, S):
    RB = B_ // NW
    M = RB * S
    GR = 128
    NG = M // GR
    RG = GR // S

    @functools.partial(
        pl.kernel,
        out_type=(
            jax.ShapeDtypeStruct((B_, H_), jnp.float32),
            jax.ShapeDtypeStruct((B_, H_), jnp.float32),
        ),
        mesh=_mesh(),
        compiler_params=_SC_PARAMS,
        scratch_types=[
            pltpu.VMEM((M,), jnp.int32),
            pltpu.VMEM((RB,), jnp.int32),
            pltpu.VMEM((2, GR, H_), jnp.float32),
            pltpu.VMEM((RB, H_), jnp.float32),
            pltpu.VMEM((RB, H_), jnp.float32),
            pltpu.SemaphoreType.DMA,
            pltpu.SemaphoreType.DMA,
        ],
    )
    def k(h_hbm, n1_hbm, cur2_hbm, sum_hbm, x_hbm, cidx, ccur, gbuf, obuf, xg,
          semA, semB):
        base = _wid() * RB
        pltpu.sync_copy(n1_hbm.at[pl.ds(base * S, M)], cidx)
        pltpu.sync_copy(cur2_hbm.at[pl.ds(base, RB)], ccur)
        sems = (semA, semB)

        _fire(h_hbm.at[ccur], xg, semA)

        def gsrc(g):
            off = pl.multiple_of(g * GR, GR)
            return h_hbm.at[cidx.at[pl.ds(off, GR)]]

        _fire(gsrc(0), gbuf.at[0], sems[1])
        _drain(h_hbm.at[ccur], xg, semA)
        pltpu.sync_copy(xg, x_hbm.at[pl.ds(base, RB)])

        for g in range(NG):
            slot = g % 2
            if g + 1 < NG:
                _fire(gsrc(g + 1), gbuf.at[1 - slot], sems[g % 2])
            _drain(gsrc(g), gbuf.at[slot], sems[1 - slot])

            def red_body(r, _):
                orow = g * RG + r
                for c in range(H_ // NL):
                    cs = pl.ds(c * NL, NL)
                    acc = gbuf[slot, r * S, cs]
                    for s in range(1, S):
                        acc = acc + gbuf[slot, r * S + s, cs]
                    obuf[orow, cs] = acc
                return 0

            lax.fori_loop(0, RG, red_body, 0)

        pltpu.sync_copy(obuf, sum_hbm.at[pl.ds(base, RB)])

    return k


def _fused_linear(x, s, Wa, Wb, blk, out_dtype):
    R, Dm = x.shape
    Hm = Wa.shape[1]

    def body(x_ref, s_ref, wa_ref, wb_ref, o_ref):
        acc = jnp.dot(x_ref[...], wa_ref[...], preferred_element_type=jnp.float32)
        acc = acc + jnp.dot(s_ref[...], wb_ref[...], preferred_element_type=jnp.float32)
        o_ref[...] = jnp.maximum(acc, 0.0).astype(out_dtype)

    return pl.pallas_call(
        body,
        grid=(R // blk,),
        in_specs=[
            pl.BlockSpec((blk, Dm), lambda i: (i, 0)),
            pl.BlockSpec((blk, Dm), lambda i: (i, 0)),
            pl.BlockSpec((Dm, Hm), lambda i: (0, 0)),
            pl.BlockSpec((Dm, Hm), lambda i: (0, 0)),
        ],
        out_specs=pl.BlockSpec((blk, Hm), lambda i: (i, 0)),
        out_shape=jax.ShapeDtypeStruct((R, Hm), out_dtype),
    )(x, s, Wa, Wb)


def _unpack_perm(D):
    p = np.empty(D, np.int32)
    for c in range(0, D, 32):
        for t in range(16):
            p[c + t] = c + 2 * t
            p[c + 16 + t] = c + 2 * t + 1
    return p


def kernel(in_features, nodes_l0, neigh2, cur1, neigh1, cur2, W0, W1):
    N_, D_ = in_features.shape
    U2 = nodes_l0.shape[0]
    U1, S = neigh2.shape
    B_ = cur2.shape[0]
    H_ = W0.shape[1]

    featw = lax.bitcast_convert_type(
        in_features.astype(jnp.bfloat16).reshape(N_, D_ // 2, 2), jnp.int32
    )
    nodes_l0 = nodes_l0.astype(jnp.int32)
    neigh2f = neigh2.astype(jnp.int32).reshape(U1 * S)
    cur1 = cur1.astype(jnp.int32)
    neigh1f = neigh1.astype(jnp.int32).reshape(B_ * S)
    cur2 = cur2.astype(jnp.int32)

    perm = jnp.asarray(_unpack_perm(D_))
    W0a = jnp.take(W0[:D_], perm, axis=0)
    W0b = jnp.take(W0[D_:] * (1.0 / S), perm, axis=0)
    W1a = W1[:H_]
    W1b = W1[H_:] * (1.0 / S)

    comp2, compc = _compose_kernel(U2, U1, S)(nodes_l0, neigh2f, cur1)
    sum0, x0 = _gather_sum_kernel(D_ // 2, U1, S)(featw, comp2, compc)
    h1 = _fused_linear(x0, sum0, W0a, W0b, 512, jnp.float32)
    sum1, x1 = _gather_sum_small_kernel(H_, B_, S)(h1, neigh1f, cur2)
    out = _fused_linear(x1, sum1, W1a, W1b, 512, jnp.float32)
    return out


# speedup vs baseline: 2.9788x; 1.8796x over previous
import functools

import jax
import jax.numpy as jnp
import numpy as np
from jax import lax
from jax.experimental import pallas as pl
from jax.experimental.pallas import tpu as pltpu
from jax.experimental.pallas import tpu_sc as plsc

NC = 2
NS = 16
NL = 16
NW = NC * NS


def _mesh():
    return plsc.VectorSubcoreMesh(
        core_axis_name="c", subcore_axis_name="s", num_cores=NC, num_subcores=NS
    )


_SC_PARAMS = pltpu.CompilerParams(needs_layout_passes=False)
_SC_PARAMS_NT = pltpu.CompilerParams(
    needs_layout_passes=False, use_tc_tiling_on_sc=False
)


def _wid():
    return lax.axis_index("s") * NC + lax.axis_index("c")


def _fire(src, dst, sem):
    pltpu.make_async_copy(src, dst, sem).start()


def _drain(src, dst, sem):
    pltpu.make_async_copy(src, dst, sem).wait()


def _compose_kernel(N_, D_, U2, U1, S):
    RPT = U1 // NW
    M = RPT * S
    W_ = D_ // 2
    RPW = N_ // NW
    CH = 125
    NCH = RPW // CH
    fmt = plsc.PackFormat.INTERLEAVED

    @functools.partial(
        pl.kernel,
        out_type=(
            jax.ShapeDtypeStruct((U1 * S,), jnp.int32),
            jax.ShapeDtypeStruct((U1,), jnp.int32),
            jax.ShapeDtypeStruct((N_, W_), jnp.int32),
        ),
        mesh=_mesh(),
        compiler_params=_SC_PARAMS_NT,
        scratch_types=[
            pltpu.SemaphoreType.DMA,
            pltpu.SemaphoreType.DMA,
            pltpu.SemaphoreType.DMA,
        ],
    )
    def k(feat_hbm, nodes_hbm, neigh2_hbm, cur1_hbm,
          comp2_hbm, compc_hbm, featw_hbm, semA, semB, semO):
        wid = _wid()
        base = wid * RPT
        fbase = wid * RPW
        fsems = (semA, semB)

        def cast_phase(fin, fout):
            def fsrc(c):
                return feat_hbm.at[pl.ds(fbase + c * CH, CH)]

            _fire(fsrc(0), fin.at[0], fsems[0])
            for c in range(NCH):
                slot = c % 2
                if c + 1 < NCH:
                    _fire(fsrc(c + 1), fin.at[1 - slot], fsems[1 - slot])
                _drain(fsrc(c), fin.at[slot], fsems[slot])
                if c >= 2:
                    _drain(fout.at[slot],
                           featw_hbm.at[pl.ds(fbase + (c - 2) * CH, CH)], semO)

                def cast_body(r, _):
                    for cc in range(W_ // 16):
                        v0 = fin[slot, r, pl.ds(cc * 32, 16)]
                        v1 = fin[slot, r, pl.ds(cc * 32 + 16, 16)]
                        w = plsc.bitcast(plsc.pack(v0, v1, format=fmt),
                                         jnp.int32)
                        fout[slot, r, pl.ds(cc * 16, 16)] = w
                    return 0

                lax.fori_loop(0, CH, cast_body, 0)
                _fire(fout.at[slot], featw_hbm.at[pl.ds(fbase + c * CH, CH)],
                      semO)
            _drain(fout.at[(NCH - 2) % 2],
                   featw_hbm.at[pl.ds(fbase + (NCH - 2) * CH, CH)], semO)
            _drain(fout.at[(NCH - 1) % 2],
                   featw_hbm.at[pl.ds(fbase + (NCH - 1) * CH, CH)], semO)

        pl.run_scoped(
            cast_phase,
            pltpu.VMEM((2, CH, D_), jnp.float32),
            pltpu.VMEM((2, CH, W_), jnp.int32),
        )

        def comp_phase(nodes_v, buf, cbuf):
            pltpu.sync_copy(nodes_hbm, nodes_v)
            pltpu.sync_copy(neigh2_hbm.at[pl.ds(base * S, M)], buf)
            pltpu.sync_copy(cur1_hbm.at[pl.ds(base, RPT)], cbuf)

            UNR = 4

            def body(i, _):
                for j in range(UNR):
                    off = pl.multiple_of(i * NL * UNR + j * NL, NL)
                    v = buf[pl.ds(off, NL)]
                    buf[pl.ds(off, NL)] = plsc.load_gather(nodes_v, [v])
                return 0

            lax.fori_loop(0, M // (NL * UNR), body, 0)

            def cbody(i, _):
                for j in range(UNR):
                    off = pl.multiple_of(i * NL * UNR + j * NL, NL)
                    v = cbuf[pl.ds(off, NL)]
                    cbuf[pl.ds(off, NL)] = plsc.load_gather(nodes_v, [v])
                return 0

            lax.fori_loop(0, RPT // (NL * UNR), cbody, 0)

            pltpu.sync_copy(buf, comp2_hbm.at[pl.ds(base * S, M)])
            pltpu.sync_copy(cbuf, compc_hbm.at[pl.ds(base, RPT)])

        pl.run_scoped(
            comp_phase,
            pltpu.VMEM((U2,), jnp.int32),
            pltpu.VMEM((M,), jnp.int32),
            pltpu.VMEM((RPT,), jnp.int32),
        )

    return k


def _gather_sum_kernel(W_, U1, S):
    D2 = 2 * W_
    RPT = U1 // NW
    GR = 128
    NG = RPT * S // GR
    RG = GR // S
    FL = 16
    OB = FL * RG
    NX = RPT // GR
    fmt = plsc.PackFormat.INTERLEAVED

    @functools.partial(
        pl.kernel,
        out_type=(
            jax.ShapeDtypeStruct((U1, D2), jnp.float32),
            jax.ShapeDtypeStruct((U1, D2), jnp.float32),
        ),
        mesh=_mesh(),
        compiler_params=_SC_PARAMS_NT,
        scratch_types=[
            pltpu.VMEM((RPT * S,), jnp.int32),
            pltpu.VMEM((RPT,), jnp.int32),
            pltpu.VMEM((2, GR, W_), jnp.int32),
            pltpu.VMEM((1, OB, D2), jnp.float32),
            pltpu.VMEM((2, GR, D2), jnp.float32),
            pltpu.SemaphoreType.DMA,
            pltpu.SemaphoreType.DMA,
            pltpu.SemaphoreType.DMA,
        ],
    )
    def k(feat_hbm, comp2_hbm, compc_hbm, sum_hbm, x_hbm, cidx, ccur, gbuf, obuf,
          xobuf, semA, semB, semX):
        base = _wid() * RPT
        pltpu.sync_copy(comp2_hbm.at[pl.ds(base * S, RPT * S)], cidx)
        pltpu.sync_copy(compc_hbm.at[pl.ds(base, RPT)], ccur)
        sems = (semA, semB)

        def xsrc(i):
            off = pl.multiple_of(i * GR, GR)
            return feat_hbm.at[ccur.at[pl.ds(off, GR)]]

        def xdst(i):
            return x_hbm.at[pl.ds(base + i * GR, GR)]

        _fire(xsrc(0), gbuf.at[0], sems[0])
        for i in range(NX):
            slot = i % 2
            if i + 1 < NX:
                _fire(xsrc(i + 1), gbuf.at[1 - slot], sems[1 - slot])
            _drain(xsrc(i), gbuf.at[slot], sems[slot])
            if i >= 2:
                _drain(xobuf.at[slot], xdst(i - 2), semX)

            def xw_body(r, _):
                for c in range(W_ // 16):
                    v = plsc.bitcast(gbuf[slot, r, pl.ds(c * 16, 16)],
                                     jnp.bfloat16)
                    a0, a1 = plsc.unpack(v, format=fmt)
                    xobuf[slot, r, pl.ds(c * 32, 16)] = a0
                    xobuf[slot, r, pl.ds(c * 32 + 16, 16)] = a1
                return 0

            lax.fori_loop(0, GR, xw_body, 0)
            _fire(xobuf.at[slot], xdst(i), semX)
        _drain(xobuf.at[(NX - 2) % 2], xdst(NX - 2), semX)
        _drain(xobuf.at[(NX - 1) % 2], xdst(NX - 1), semX)

        def gsrc(g):
            off = pl.multiple_of(g * GR, GR)
            return feat_hbm.at[cidx.at[pl.ds(off, GR)]]

        def gstart(g, slot):
            _fire(gsrc(g), gbuf.at[slot], sems[slot])

        gstart(0, 0)

        def pair_body(gp, _):
            for kk in range(2):
                g = gp * 2 + kk
                slot = kk

                @pl.when(g + 1 < NG)
                def _():
                    gstart(g + 1, 1 - slot)

                _drain(gsrc(g), gbuf.at[slot], sems[slot])

                def red_body(r, _):
                    orow = (g % FL) * RG + r
                    for c in range(W_ // 16):
                        cs = pl.ds(c * 16, 16)
                        v = plsc.bitcast(gbuf[slot, r * S, cs], jnp.bfloat16)
                        a0, a1 = plsc.unpack(v, format=fmt)
                        for s in range(1, S):
                            v = plsc.bitcast(gbuf[slot, r * S + s, cs],
                                             jnp.bfloat16)
                            b0, b1 = plsc.unpack(v, format=fmt)
                            a0 = a0 + b0
                            a1 = a1 + b1
                        obuf[0, orow, pl.ds(c * 32, 16)] = a0
                        obuf[0, orow, pl.ds(c * 32 + 16, 16)] = a1
                    return 0

                lax.fori_loop(0, RG, red_body, 0)

                @pl.when(g % FL == FL - 1)
                def _():
                    pltpu.sync_copy(
                        obuf.at[0], sum_hbm.at[pl.ds(base + (g // FL) * OB, OB)]
                    )
            return 0

        lax.fori_loop(0, NG // 2, pair_body, 0)

    return k


def _gather_sum_small_kernel(H_, B_, S):
    RB = B_ // NW
    M = RB * S
    GR = 128
    NG = M // GR
    RG = GR // S

    @functools.partial(
        pl.kernel,
        out_type=(
            jax.ShapeDtypeStruct((B_, H_), jnp.float32),
            jax.ShapeDtypeStruct((B_, H_), jnp.float32),
        ),
        mesh=_mesh(),
        compiler_params=_SC_PARAMS,
        scratch_types=[
            pltpu.VMEM((M,), jnp.int32),
            pltpu.VMEM((RB,), jnp.int32),
            pltpu.VMEM((2, GR, H_), jnp.float32),
            pltpu.VMEM((RB, H_), jnp.float32),
            pltpu.VMEM((RB, H_), jnp.float32),
            pltpu.SemaphoreType.DMA,
            pltpu.SemaphoreType.DMA,
        ],
    )
    def k(h_hbm, n1_hbm, cur2_hbm, sum_hbm, x_hbm, cidx, ccur, gbuf, obuf, xg,
          semA, semB):
        base = _wid() * RB
        pltpu.sync_copy(n1_hbm.at[pl.ds(base * S, M)], cidx)
        pltpu.sync_copy(cur2_hbm.at[pl.ds(base, RB)], ccur)
        sems = (semA, semB)

        _fire(h_hbm.at[ccur], xg, semA)

        def gsrc(g):
            off = pl.multiple_of(g * GR, GR)
            return h_hbm.at[cidx.at[pl.ds(off, GR)]]

        _fire(gsrc(0), gbuf.at[0], sems[1])
        _drain(h_hbm.at[ccur], xg, semA)
        pltpu.sync_copy(xg, x_hbm.at[pl.ds(base, RB)])

        for g in range(NG):
            slot = g % 2
            if g + 1 < NG:
                _fire(gsrc(g + 1), gbuf.at[1 - slot], sems[g % 2])
            _drain(gsrc(g), gbuf.at[slot], sems[1 - slot])

            def red_body(r, _):
                orow = g * RG + r
                for c in range(H_ // NL):
                    cs = pl.ds(c * NL, NL)
                    acc = gbuf[slot, r * S, cs]
                    for s in range(1, S):
                        acc = acc + gbuf[slot, r * S + s, cs]
                    obuf[orow, cs] = acc
                return 0

            lax.fori_loop(0, RG, red_body, 0)

        pltpu.sync_copy(obuf, sum_hbm.at[pl.ds(base, RB)])

    return k


def _fused_linear(x, s, Wa, Wb, blk, out_dtype):
    R, Dm = x.shape
    Hm = Wa.shape[1]

    def body(x_ref, s_ref, wa_ref, wb_ref, o_ref):
        acc = jnp.dot(x_ref[...], wa_ref[...], preferred_element_type=jnp.float32)
        acc = acc + jnp.dot(s_ref[...], wb_ref[...], preferred_element_type=jnp.float32)
        o_ref[...] = jnp.maximum(acc, 0.0).astype(out_dtype)

    return pl.pallas_call(
        body,
        grid=(R // blk,),
        in_specs=[
            pl.BlockSpec((blk, Dm), lambda i: (i, 0)),
            pl.BlockSpec((blk, Dm), lambda i: (i, 0)),
            pl.BlockSpec((Dm, Hm), lambda i: (0, 0)),
            pl.BlockSpec((Dm, Hm), lambda i: (0, 0)),
        ],
        out_specs=pl.BlockSpec((blk, Hm), lambda i: (i, 0)),
        out_shape=jax.ShapeDtypeStruct((R, Hm), out_dtype),
    )(x, s, Wa, Wb)


def kernel(in_features, nodes_l0, neigh2, cur1, neigh1, cur2, W0, W1):
    N_, D_ = in_features.shape
    U2 = nodes_l0.shape[0]
    U1, S = neigh2.shape
    B_ = cur2.shape[0]
    H_ = W0.shape[1]

    nodes_l0 = nodes_l0.astype(jnp.int32)
    neigh2f = neigh2.astype(jnp.int32).reshape(U1 * S)
    cur1 = cur1.astype(jnp.int32)
    neigh1f = neigh1.astype(jnp.int32).reshape(B_ * S)
    cur2 = cur2.astype(jnp.int32)

    W0a = W0[:D_]
    W0b = W0[D_:] * (1.0 / S)
    W1a = W1[:H_]
    W1b = W1[H_:] * (1.0 / S)

    comp2, compc, featw = _compose_kernel(N_, D_, U2, U1, S)(
        in_features, nodes_l0, neigh2f, cur1)
    sum0, x0 = _gather_sum_kernel(D_ // 2, U1, S)(featw, comp2, compc)
    h1 = _fused_linear(x0, sum0, W0a, W0b, 512, jnp.float32)
    sum1, x1 = _gather_sum_small_kernel(H_, B_, S)(h1, neigh1f, cur2)
    out = _fused_linear(x1, sum1, W1a, W1b, 512, jnp.float32)
    return out


# speedup vs baseline: 3.2449x; 1.0893x over previous
import functools

import jax
import jax.numpy as jnp
import numpy as np
from jax import lax
from jax.experimental import pallas as pl
from jax.experimental.pallas import tpu as pltpu
from jax.experimental.pallas import tpu_sc as plsc

NC = 2
NS = 16
NL = 16
NW = NC * NS


def _mesh():
    return plsc.VectorSubcoreMesh(
        core_axis_name="c", subcore_axis_name="s", num_cores=NC, num_subcores=NS
    )


_SC_PARAMS = pltpu.CompilerParams(needs_layout_passes=False)
_SC_PARAMS_NT = pltpu.CompilerParams(
    needs_layout_passes=False, use_tc_tiling_on_sc=False
)


def _wid():
    return lax.axis_index("s") * NC + lax.axis_index("c")


def _fire(src, dst, sem):
    pltpu.make_async_copy(src, dst, sem).start()


def _drain(src, dst, sem):
    pltpu.make_async_copy(src, dst, sem).wait()


def _compose_kernel(N_, D_, U2, U1, S):
    RPT = U1 // NW
    M = RPT * S
    W_ = D_ // 2
    RPW = N_ // NW
    CH = 125
    NCH = RPW // CH
    fmt = plsc.PackFormat.INTERLEAVED

    @functools.partial(
        pl.kernel,
        out_type=(
            jax.ShapeDtypeStruct((U1 * S,), jnp.int32),
            jax.ShapeDtypeStruct((U1,), jnp.int32),
            jax.ShapeDtypeStruct((N_, W_), jnp.int32),
        ),
        mesh=_mesh(),
        compiler_params=_SC_PARAMS_NT,
        scratch_types=[
            pltpu.SemaphoreType.DMA,
            pltpu.SemaphoreType.DMA,
            pltpu.SemaphoreType.DMA,
        ],
    )
    def k(feat_hbm, nodes_hbm, neigh2_hbm, cur1_hbm,
          comp2_hbm, compc_hbm, featw_hbm, semA, semB, semO):
        wid = _wid()
        base = wid * RPT
        fbase = wid * RPW
        fsems = (semA, semB)

        def cast_phase(fin, fout):
            def fsrc(c):
                return feat_hbm.at[pl.ds(fbase + c * CH, CH)]

            _fire(fsrc(0), fin.at[0], fsems[0])
            for c in range(NCH):
                slot = c % 2
                if c + 1 < NCH:
                    _fire(fsrc(c + 1), fin.at[1 - slot], fsems[1 - slot])
                _drain(fsrc(c), fin.at[slot], fsems[slot])
                if c >= 2:
                    _drain(fout.at[slot],
                           featw_hbm.at[pl.ds(fbase + (c - 2) * CH, CH)], semO)

                def cast_body(r, _):
                    for cc in range(W_ // 16):
                        v0 = fin[slot, r, pl.ds(cc * 32, 16)]
                        v1 = fin[slot, r, pl.ds(cc * 32 + 16, 16)]
                        w = plsc.bitcast(plsc.pack(v0, v1, format=fmt),
                                         jnp.int32)
                        fout[slot, r, pl.ds(cc * 16, 16)] = w
                    return 0

                lax.fori_loop(0, CH, cast_body, 0)
                _fire(fout.at[slot], featw_hbm.at[pl.ds(fbase + c * CH, CH)],
                      semO)
            _drain(fout.at[(NCH - 2) % 2],
                   featw_hbm.at[pl.ds(fbase + (NCH - 2) * CH, CH)], semO)
            _drain(fout.at[(NCH - 1) % 2],
                   featw_hbm.at[pl.ds(fbase + (NCH - 1) * CH, CH)], semO)

        pl.run_scoped(
            cast_phase,
            pltpu.VMEM((2, CH, D_), jnp.float32),
            pltpu.VMEM((2, CH, W_), jnp.int32),
        )

        def comp_phase(nodes_v, buf, cbuf):
            pltpu.sync_copy(nodes_hbm, nodes_v)
            pltpu.sync_copy(neigh2_hbm.at[pl.ds(base * S, M)], buf)
            pltpu.sync_copy(cur1_hbm.at[pl.ds(base, RPT)], cbuf)

            UNR = 4

            def body(i, _):
                for j in range(UNR):
                    off = pl.multiple_of(i * NL * UNR + j * NL, NL)
                    v = buf[pl.ds(off, NL)]
                    buf[pl.ds(off, NL)] = plsc.load_gather(nodes_v, [v])
                return 0

            lax.fori_loop(0, M // (NL * UNR), body, 0)

            def cbody(i, _):
                for j in range(UNR):
                    off = pl.multiple_of(i * NL * UNR + j * NL, NL)
                    v = cbuf[pl.ds(off, NL)]
                    cbuf[pl.ds(off, NL)] = plsc.load_gather(nodes_v, [v])
                return 0

            lax.fori_loop(0, RPT // (NL * UNR), cbody, 0)

            pltpu.sync_copy(buf, comp2_hbm.at[pl.ds(base * S, M)])
            pltpu.sync_copy(cbuf, compc_hbm.at[pl.ds(base, RPT)])

        pl.run_scoped(
            comp_phase,
            pltpu.VMEM((U2,), jnp.int32),
            pltpu.VMEM((M,), jnp.int32),
            pltpu.VMEM((RPT,), jnp.int32),
        )

    return k


def _gather_sum_kernel(W_, U1, S):
    D2 = 2 * W_
    RPT = U1 // NW
    GR = 128
    NG = RPT * S // GR
    RG = GR // S
    FL = 8
    OB = FL * RG
    NX = RPT // GR
    fmt = plsc.PackFormat.INTERLEAVED

    @functools.partial(
        pl.kernel,
        out_type=(
            jax.ShapeDtypeStruct((U1, D2), jnp.float32),
            jax.ShapeDtypeStruct((U1, D2), jnp.float32),
        ),
        mesh=_mesh(),
        compiler_params=_SC_PARAMS_NT,
        scratch_types=[
            pltpu.VMEM((RPT * S,), jnp.int32),
            pltpu.VMEM((RPT,), jnp.int32),
            pltpu.VMEM((4, GR, W_), jnp.int32),
            pltpu.VMEM((2, OB, D2), jnp.float32),
            pltpu.VMEM((2, GR, D2), jnp.float32),
            pltpu.SemaphoreType.DMA,
            pltpu.SemaphoreType.DMA,
            pltpu.SemaphoreType.DMA,
            pltpu.SemaphoreType.DMA,
            pltpu.SemaphoreType.DMA,
            pltpu.SemaphoreType.DMA,
            pltpu.SemaphoreType.DMA,
        ],
    )
    def k(feat_hbm, comp2_hbm, compc_hbm, sum_hbm, x_hbm, cidx, ccur, gbuf, obuf,
          xobuf, semA, semB, semC, semD, fA, fB, semX):
        base = _wid() * RPT
        pltpu.sync_copy(comp2_hbm.at[pl.ds(base * S, RPT * S)], cidx)
        pltpu.sync_copy(compc_hbm.at[pl.ds(base, RPT)], ccur)
        sems = (semA, semB, semC, semD)

        def xsrc(i):
            off = pl.multiple_of(i * GR, GR)
            return feat_hbm.at[ccur.at[pl.ds(off, GR)]]

        def xdst(i):
            return x_hbm.at[pl.ds(base + i * GR, GR)]

        _fire(xsrc(0), gbuf.at[0], sems[0])
        for i in range(NX):
            slot = i % 2
            if i + 1 < NX:
                _fire(xsrc(i + 1), gbuf.at[1 - slot], sems[1 - slot])
            _drain(xsrc(i), gbuf.at[slot], sems[slot])
            if i >= 2:
                _drain(xobuf.at[slot], xdst(i - 2), semX)

            def xw_body(r, _):
                for c in range(W_ // 16):
                    v = plsc.bitcast(gbuf[slot, r, pl.ds(c * 16, 16)],
                                     jnp.bfloat16)
                    a0, a1 = plsc.unpack(v, format=fmt)
                    xobuf[slot, r, pl.ds(c * 32, 16)] = a0
                    xobuf[slot, r, pl.ds(c * 32 + 16, 16)] = a1
                return 0

            lax.fori_loop(0, GR, xw_body, 0)
            _fire(xobuf.at[slot], xdst(i), semX)
        _drain(xobuf.at[(NX - 2) % 2], xdst(NX - 2), semX)
        _drain(xobuf.at[(NX - 1) % 2], xdst(NX - 1), semX)

        def gsrc(g):
            off = pl.multiple_of(g * GR, GR)
            return feat_hbm.at[cidx.at[pl.ds(off, GR)]]

        def gstart(g, slot):
            _fire(gsrc(g), gbuf.at[slot], sems[slot])

        def obuf_dst(g):
            return sum_hbm.at[pl.ds(base + (g // FL) * OB, OB)]

        gstart(0, 0)
        gstart(1, 1)
        fsems = (fA, fB)

        def quad_body(gq, _):
            for kk in range(4):
                g = gq * 4 + kk
                slot = kk
                obs = (g // FL) % 2

                @pl.when(g + 2 < NG)
                def _():
                    gstart(g + 2, (kk + 2) % 4)

                if kk == 0:
                    @pl.when((g % FL == 0) & (g >= 2 * FL) & (obs == 0))
                    def _():
                        _drain(obuf.at[0], obuf_dst(g - 2 * FL), fA)

                    @pl.when((g % FL == 0) & (g >= 2 * FL) & (obs == 1))
                    def _():
                        _drain(obuf.at[1], obuf_dst(g - 2 * FL), fB)

                _drain(gsrc(g), gbuf.at[slot], sems[slot])

                def red_body(r, _):
                    orow = (g % FL) * RG + r
                    for c in range(W_ // 16):
                        cs = pl.ds(c * 16, 16)
                        v = plsc.bitcast(gbuf[slot, r * S, cs], jnp.bfloat16)
                        a0, a1 = plsc.unpack(v, format=fmt)
                        for s in range(1, S):
                            v = plsc.bitcast(gbuf[slot, r * S + s, cs],
                                             jnp.bfloat16)
                            b0, b1 = plsc.unpack(v, format=fmt)
                            a0 = a0 + b0
                            a1 = a1 + b1
                        obuf[obs, orow, pl.ds(c * 32, 16)] = a0
                        obuf[obs, orow, pl.ds(c * 32 + 16, 16)] = a1
                    return 0

                lax.fori_loop(0, RG, red_body, 0)

                if kk == 3:
                    @pl.when((g % FL == FL - 1) & (obs == 0))
                    def _():
                        _fire(obuf.at[0], obuf_dst(g), fA)

                    @pl.when((g % FL == FL - 1) & (obs == 1))
                    def _():
                        _fire(obuf.at[1], obuf_dst(g), fB)
            return 0

        lax.fori_loop(0, NG // 4, quad_body, 0)

        _drain(obuf.at[0], obuf_dst(NG - 2 * FL), fA)
        _drain(obuf.at[1], obuf_dst(NG - FL), fB)

    return k


def _gather_sum_small_kernel(H_, B_, S):
    RB = B_ // NW
    M = RB * S
    GR = 128
    NG = M // GR
    RG = GR // S

    @functools.partial(
        pl.kernel,
        out_type=(
            jax.ShapeDtypeStruct((B_, H_), jnp.float32),
            jax.ShapeDtypeStruct((B_, H_), jnp.float32),
        ),
        mesh=_mesh(),
        compiler_params=_SC_PARAMS,
        scratch_types=[
            pltpu.VMEM((M,), jnp.int32),
            pltpu.VMEM((RB,), jnp.int32),
            pltpu.VMEM((2, GR, H_), jnp.float32),
            pltpu.VMEM((RB, H_), jnp.float32),
            pltpu.VMEM((RB, H_), jnp.float32),
            pltpu.SemaphoreType.DMA,
            pltpu.SemaphoreType.DMA,
        ],
    )
    def k(h_hbm, n1_hbm, cur2_hbm, sum_hbm, x_hbm, cidx, ccur, gbuf, obuf, xg,
          semA, semB):
        base = _wid() * RB
        pltpu.sync_copy(n1_hbm.at[pl.ds(base * S, M)], cidx)
        pltpu.sync_copy(cur2_hbm.at[pl.ds(base, RB)], ccur)
        sems = (semA, semB)

        _fire(h_hbm.at[ccur], xg, semA)

        def gsrc(g):
            off = pl.multiple_of(g * GR, GR)
            return h_hbm.at[cidx.at[pl.ds(off, GR)]]

        _fire(gsrc(0), gbuf.at[0], sems[1])
        _drain(h_hbm.at[ccur], xg, semA)
        pltpu.sync_copy(xg, x_hbm.at[pl.ds(base, RB)])

        for g in range(NG):
            slot = g % 2
            if g + 1 < NG:
                _fire(gsrc(g + 1), gbuf.at[1 - slot], sems[g % 2])
            _drain(gsrc(g), gbuf.at[slot], sems[1 - slot])

            def red_body(r, _):
                orow = g * RG + r
                for c in range(H_ // NL):
                    cs = pl.ds(c * NL, NL)
                    acc = gbuf[slot, r * S, cs]
                    for s in range(1, S):
                        acc = acc + gbuf[slot, r * S + s, cs]
                    obuf[orow, cs] = acc
                return 0

            lax.fori_loop(0, RG, red_body, 0)

        pltpu.sync_copy(obuf, sum_hbm.at[pl.ds(base, RB)])

    return k


def _fused_linear(x, s, Wa, Wb, blk, out_dtype):
    R, Dm = x.shape
    Hm = Wa.shape[1]

    def body(x_ref, s_ref, wa_ref, wb_ref, o_ref):
        acc = jnp.dot(x_ref[...], wa_ref[...], preferred_element_type=jnp.float32)
        acc = acc + jnp.dot(s_ref[...], wb_ref[...], preferred_element_type=jnp.float32)
        o_ref[...] = jnp.maximum(acc, 0.0).astype(out_dtype)

    return pl.pallas_call(
        body,
        grid=(R // blk,),
        in_specs=[
            pl.BlockSpec((blk, Dm), lambda i: (i, 0)),
            pl.BlockSpec((blk, Dm), lambda i: (i, 0)),
            pl.BlockSpec((Dm, Hm), lambda i: (0, 0)),
            pl.BlockSpec((Dm, Hm), lambda i: (0, 0)),
        ],
        out_specs=pl.BlockSpec((blk, Hm), lambda i: (i, 0)),
        out_shape=jax.ShapeDtypeStruct((R, Hm), out_dtype),
    )(x, s, Wa, Wb)


def kernel(in_features, nodes_l0, neigh2, cur1, neigh1, cur2, W0, W1):
    N_, D_ = in_features.shape
    U2 = nodes_l0.shape[0]
    U1, S = neigh2.shape
    B_ = cur2.shape[0]
    H_ = W0.shape[1]

    nodes_l0 = nodes_l0.astype(jnp.int32)
    neigh2f = neigh2.astype(jnp.int32).reshape(U1 * S)
    cur1 = cur1.astype(jnp.int32)
    neigh1f = neigh1.astype(jnp.int32).reshape(B_ * S)
    cur2 = cur2.astype(jnp.int32)

    W0a = W0[:D_]
    W0b = W0[D_:] * (1.0 / S)
    W1a = W1[:H_]
    W1b = W1[H_:] * (1.0 / S)

    comp2, compc, featw = _compose_kernel(N_, D_, U2, U1, S)(
        in_features, nodes_l0, neigh2f, cur1)
    sum0, x0 = _gather_sum_kernel(D_ // 2, U1, S)(featw, comp2, compc)
    h1 = _fused_linear(x0, sum0, W0a, W0b, 512, jnp.float32)
    sum1, x1 = _gather_sum_small_kernel(H_, B_, S)(h1, neigh1f, cur2)
    out = _fused_linear(x1, sum1, W1a, W1b, 512, jnp.float32)
    return out
